# Initial kernel scaffold; baseline (speedup 1.0000x reference)
import jax
import jax.numpy as jnp
from jax.experimental import pallas as pl


def kernel(input_ids, token_type_ids, word_embeddings, position_embeddings, token_type_embeddings, gamma, beta):
    raise NotImplementedError("write your pallas kernel here")


# speedup vs baseline: 1.7103x; 1.7103x over previous
import functools

import jax
import jax.numpy as jnp
from jax import lax
from jax.experimental import pallas as pl
from jax.experimental.pallas import tpu as pltpu
from jax.experimental.pallas import tpu_sc as plsc

_EPS = 1e-12
_LANES = 16


_GATHER_DNUMS = lax.GatherDimensionNumbers(
    offset_dims=(), collapsed_slice_dims=(0,), start_index_map=(0,))


def _shuffle(x, k):
    perm = lax.iota(jnp.int32, _LANES) ^ k
    return lax.gather(x, perm[:, None], _GATHER_DNUMS, (1,),
                      mode=lax.GatherScatterMode.PROMISE_IN_BOUNDS)


def _allsum(x):
    for k in (8, 4, 2, 1):
        x = x + _shuffle(x, k)
    return x


def _rsqrt(v16):
    i = lax.bitcast_convert_type(v16, jnp.int32)
    y = lax.bitcast_convert_type(jnp.int32(0x5F3759DF) - (i >> 1), jnp.float32)
    for _ in range(3):
        y = y * (1.5 - 0.5 * v16 * y * y)
    return y


def _make_sc_kernel(N, S, H, V, C, n_workers):
    tpw = N // n_workers
    n_chunks = tpw // C
    hs = H // _LANES

    mesh = plsc.VectorSubcoreMesh(core_axis_name="c", subcore_axis_name="s")

    @functools.partial(
        pl.kernel,
        out_type=jax.ShapeDtypeStruct((N, H), jnp.float32),
        mesh=mesh,
        scratch_types=[
            pltpu.VMEM((C,), jnp.int32),
            pltpu.VMEM((C,), jnp.int32),
            pltpu.VMEM((C, H), jnp.float32),
            pltpu.VMEM((S, H), jnp.float32),
            pltpu.VMEM((2, H), jnp.float32),
            pltpu.VMEM((H,), jnp.float32),
            pltpu.VMEM((H,), jnp.float32),
            pltpu.SemaphoreType.DMA,
        ],
    )
    def sc_kernel(ids_hbm, tts_hbm, w_hbm, p_hbm, t_hbm, g_hbm, b_hbm,
                  out_hbm, idx_v, tt_v, rows_v, pos_v, ttab_v, g_v, b_v, sem):
        info = plsc.get_sparse_core_info()
        wid = lax.axis_index("s") * info.num_cores + lax.axis_index("c")
        base_w = wid * tpw

        pltpu.sync_copy(p_hbm.at[pl.ds(0, S)], pos_v)
        pltpu.sync_copy(t_hbm, ttab_v)
        pltpu.sync_copy(g_hbm, g_v)
        pltpu.sync_copy(b_hbm, b_v)

        def chunk_body(ci, carry):
            base = base_w + ci * C
            pltpu.sync_copy(ids_hbm.at[pl.ds(base, C)], idx_v)
            pltpu.sync_copy(tts_hbm.at[pl.ds(base, C)], tt_v)
            pltpu.async_copy(w_hbm.at[idx_v], rows_v, sem).wait()

            def grp_body(g, carry2):
                tg = g * _LANES
                mf16 = tt_v[pl.ds(tg, _LANES)].astype(jnp.float32)
                for j in range(_LANES):
                    t = tg + j
                    s = lax.rem(t, S)
                    mf = mf16[j]
                    acc = jnp.zeros((_LANES,), jnp.float32)
                    acc2 = jnp.zeros((_LANES,), jnp.float32)
                    xs = []
                    for h in range(hs):
                        sl = pl.ds(h * _LANES, _LANES)
                        t0 = ttab_v[0, sl]
                        t1 = ttab_v[1, sl]
                        x = rows_v[t, sl] + pos_v[s, sl] + t0 + mf * (t1 - t0)
                        xs.append(x)
                        acc = acc + x
                        acc2 = acc2 + x * x
                    mean = _allsum(acc) * (1.0 / H)
                    var = _allsum(acc2) * (1.0 / H) - mean * mean
                    rstd = _rsqrt(var + _EPS)
                    for h in range(hs):
                        sl = pl.ds(h * _LANES, _LANES)
                        rows_v[t, sl] = (xs[h] - mean) * rstd * g_v[sl] + b_v[sl]
                return carry2

            lax.fori_loop(0, C // _LANES, grp_body, 0)
            pltpu.sync_copy(rows_v, out_hbm.at[pl.ds(base, C)])
            return carry

        lax.fori_loop(0, n_chunks, chunk_body, 0)

    return sc_kernel


def kernel(input_ids, token_type_ids, word_embeddings, position_embeddings,
           token_type_embeddings, gamma, beta):
    B, S = input_ids.shape
    V, H = word_embeddings.shape
    N = B * S
    info = plsc.get_sparse_core_info()
    n_workers = info.num_cores * info.num_subcores
    C = 400
    sc = _make_sc_kernel(N, S, H, V, C, n_workers)
    out = sc(
        input_ids.reshape(N),
        token_type_ids.reshape(N),
        word_embeddings,
        position_embeddings,
        token_type_embeddings,
        gamma,
        beta,
    )
    return out.reshape(B, S, H)


# speedup vs baseline: 5.9474x; 3.4773x over previous
import functools

import jax
import jax.numpy as jnp
from jax import lax
from jax.experimental import pallas as pl
from jax.experimental.pallas import tpu as pltpu
from jax.experimental.pallas import tpu_sc as plsc

_EPS = 1e-12
_LANES = 16

_GATHER_DNUMS = lax.GatherDimensionNumbers(
    offset_dims=(), collapsed_slice_dims=(0,), start_index_map=(0,))


def _shuffle(x, k):
    perm = lax.iota(jnp.int32, _LANES) ^ k
    return lax.gather(x, perm[:, None], _GATHER_DNUMS, (1,),
                      mode=lax.GatherScatterMode.PROMISE_IN_BOUNDS)


def _allsum(x):
    for k in (8, 4, 2, 1):
        x = x + _shuffle(x, k)
    return x


def _rsqrt(v16):
    i = lax.bitcast_convert_type(v16, jnp.int32)
    y = lax.bitcast_convert_type(jnp.int32(0x5F3759DF) - (i >> 1), jnp.float32)
    for _ in range(2):
        y = y * (1.5 - 0.5 * v16 * y * y)
    return y


def _make_sc_kernel(N, S, H, V, C, n_workers):
    tpw = N // n_workers
    n_chunks = tpw // C
    n_half = n_chunks // 2
    n_grp = C // _LANES
    hs = H // _LANES

    mesh = plsc.VectorSubcoreMesh(core_axis_name="c", subcore_axis_name="s")

    @functools.partial(
        pl.kernel,
        out_type=jax.ShapeDtypeStruct((N, H), jnp.float32),
        mesh=mesh,
        scratch_types=[
            pltpu.VMEM((C,), jnp.int32), pltpu.VMEM((C,), jnp.int32),
            pltpu.VMEM((C,), jnp.int32), pltpu.VMEM((C,), jnp.int32),
            pltpu.VMEM((C, H), jnp.float32), pltpu.VMEM((C, H), jnp.float32),
            pltpu.VMEM((C, H), jnp.float32), pltpu.VMEM((C, H), jnp.float32),
            pltpu.VMEM((S, H), jnp.float32),
            pltpu.VMEM((2, H), jnp.float32),
            pltpu.VMEM((H,), jnp.float32),
            pltpu.VMEM((H,), jnp.float32),
            pltpu.SemaphoreType.DMA, pltpu.SemaphoreType.DMA,
            pltpu.SemaphoreType.DMA, pltpu.SemaphoreType.DMA,
        ],
    )
    def sc_kernel(ids_hbm, tts_hbm, w_hbm, p_hbm, t_hbm, g_hbm, b_hbm,
                  out_hbm, idx0, idx1, tt0, tt1, rows0, rows1, out0, out1,
                  pos_v, ttab_v, g_v, b_v, sem_g0, sem_g1, sem_w0, sem_w1):
        info = plsc.get_sparse_core_info()
        wid = lax.axis_index("s") * info.num_cores + lax.axis_index("c")
        base_w = wid * tpw

        pltpu.sync_copy(p_hbm.at[pl.ds(0, S)], pos_v)
        pltpu.sync_copy(t_hbm, ttab_v)
        pltpu.sync_copy(g_hbm, g_v)
        pltpu.sync_copy(b_hbm, b_v)

        def fold_body(s, carry):
            for h in range(hs):
                sl = pl.ds(h * _LANES, _LANES)
                pos_v[s, sl] = pos_v[s, sl] + ttab_v[0, sl]
            return carry
        lax.fori_loop(0, S, fold_body, 0)

        d8, g8, b8 = [], [], []
        for h in range(hs):
            sl = pl.ds(h * _LANES, _LANES)
            d8.append(ttab_v[1, sl] - ttab_v[0, sl])
            g8.append(g_v[sl])
            b8.append(b_v[sl])

        def copy_ids(ci, idx, tt):
            pltpu.sync_copy(ids_hbm.at[pl.ds(base_w + ci * C, C)], idx)
            pltpu.sync_copy(tts_hbm.at[pl.ds(base_w + ci * C, C)], tt)

        def gather(idx, rows, sem):
            return pltpu.make_async_copy(w_hbm.at[idx], rows, sem)

        def writeback(ci, out, sem):
            return pltpu.make_async_copy(
                out, out_hbm.at[pl.ds(base_w + ci * C, C)], sem)

        def compute(ci, rows, tt, out):
            def grp_body(gi, carry):
                tg = gi * _LANES
                mf16 = tt[pl.ds(tg, _LANES)].astype(jnp.float32)
                sg = lax.rem(ci * C + tg, S)
                for j in range(_LANES):
                    t = tg + j
                    s = lax.rem(sg + j, S)
                    mf = mf16[j]
                    acc = jnp.zeros((_LANES,), jnp.float32)
                    acc2 = jnp.zeros((_LANES,), jnp.float32)
                    xs = []
                    for h in range(hs):
                        sl = pl.ds(h * _LANES, _LANES)
                        x = rows[t, sl] + pos_v[s, sl] + mf * d8[h]
                        xs.append(x)
                        acc = acc + x
                        acc2 = acc2 + x * x
                    mean = _allsum(acc) * (1.0 / H)
                    var = _allsum(acc2) * (1.0 / H) - mean * mean
                    rstd = _rsqrt(var + _EPS)
                    for h in range(hs):
                        sl = pl.ds(h * _LANES, _LANES)
                        out[t, sl] = (xs[h] - mean) * rstd * g8[h] + b8[h]
                return carry
            lax.fori_loop(0, n_grp, grp_body, 0)

        copy_ids(0, idx0, tt0)
        g0 = gather(idx0, rows0, sem_g0)
        g0.start()
        g0.wait()

        def pipe_body(it, carry):
            ci0 = it * 2
            ci1 = ci0 + 1

            copy_ids(ci1, idx1, tt1)
            gather(idx1, rows1, sem_g1).start()

            @pl.when(it > 0)
            def _():
                writeback(ci0, out0, sem_w0).wait()

            compute(ci0, rows0, tt0, out0)
            writeback(ci0, out0, sem_w0).start()
            gather(idx1, rows1, sem_g1).wait()

            @pl.when(it < n_half - 1)
            def _():
                copy_ids(ci0 + 2, idx0, tt0)
                gather(idx0, rows0, sem_g0).start()

            @pl.when(it > 0)
            def _():
                writeback(ci1, out1, sem_w1).wait()

            compute(ci1, rows1, tt1, out1)
            writeback(ci1, out1, sem_w1).start()

            @pl.when(it < n_half - 1)
            def _():
                gather(idx0, rows0, sem_g0).wait()

            return carry

        lax.fori_loop(0, n_half, pipe_body, 0)

        writeback(n_chunks - 2, out0, sem_w0).wait()
        writeback(n_chunks - 1, out1, sem_w1).wait()

    return sc_kernel


def kernel(input_ids, token_type_ids, word_embeddings, position_embeddings,
           token_type_embeddings, gamma, beta):
    B, S = input_ids.shape
    V, H = word_embeddings.shape
    N = B * S
    info = plsc.get_sparse_core_info()
    n_workers = info.num_cores * info.num_subcores
    C = 160
    sc = _make_sc_kernel(N, S, H, V, C, n_workers)
    out = sc(
        input_ids.reshape(N),
        token_type_ids.reshape(N),
        word_embeddings,
        position_embeddings,
        token_type_embeddings,
        gamma,
        beta,
    )
    return out.reshape(B, S, H)
